# baseline (device time: 537714 ns/iter reference)
import jax
import jax.numpy as jnp
from jax import lax
from jax.experimental import pallas as pl
from jax.experimental.pallas import tpu as pltpu

NC = 32
SPLIT = 18
S = 3
RB = 6
LEAD = 3


def kernel(x):
    m, n = x.shape
    r = m // NC

    def body(x_ref, out_ref, stage, ring, in_sems, out_sems,
             x_send_sems, x_recv_sems, y_send_sems, y_recv_sems):
        my_x = lax.axis_index("x")
        my_y = lax.axis_index("y")
        other_x = 1 - my_x
        other_y = 1 - my_y
        base_mine = my_x * m
        base_rem = other_x * m

        ns = SPLIT - (2 * SPLIT - NC) * my_y
        ns_o = SPLIT - (2 * SPLIT - NC) * other_y

        def chunk_id(o, y):
            return lax.rem(o + SPLIT * y, NC)

        barrier_sem = pltpu.get_barrier_semaphore()
        for nbr in [(other_x, my_y), (my_x, other_y)]:
            pl.semaphore_signal(
                barrier_sem, inc=1,
                device_id=nbr, device_id_type=pl.DeviceIdType.MESH,
            )
        pl.semaphore_wait(barrier_sem, 2)

        def stage_in(c):
            cp = pltpu.make_async_copy(
                x_ref.at[pl.ds(chunk_id(c, my_y) * r, r)],
                stage.at[c % S],
                in_sems.at[c % S],
            )
            cp.start()
            return cp

        ins = [None] * NC
        outs = [None] * NC
        x_sends = [None] * SPLIT
        y_sends = [None] * SPLIT
        for c in range(S):
            ins[c] = stage_in(c)

        for c in range(NC):
            cid = chunk_id(c, my_y)
            ins[c].wait()
            if c >= RB:
                outs[c - RB].wait()
                if c - RB < SPLIT:
                    @pl.when((c - RB) < ns)
                    def _(c=c):
                        x_sends[c - RB].wait_send()
            ring[c % RB] = stage[c % S].astype(jnp.bfloat16)
            nxt = c + S
            if nxt < NC:
                ins[nxt] = stage_in(nxt)
            outs[c] = pltpu.make_async_copy(
                ring.at[c % RB],
                out_ref.at[pl.ds(base_mine + cid * r, r)],
                out_sems.at[c],
            )
            outs[c].start()
            if c < SPLIT:
                s = pltpu.make_async_remote_copy(
                    src_ref=ring.at[c % RB],
                    dst_ref=out_ref.at[pl.ds(base_mine + cid * r, r)],
                    send_sem=x_send_sems.at[c],
                    recv_sem=x_recv_sems.at[c],
                    device_id=(other_x, my_y),
                    device_id_type=pl.DeviceIdType.MESH,
                )
                x_sends[c] = s

                @pl.when(c < ns)
                def _(s=s):
                    s.start()

            o = c - LEAD
            if 0 <= o < SPLIT:
                rows = pl.ds(base_rem + chunk_id(o, my_y) * r, r)
                rcv = pltpu.make_async_remote_copy(
                    src_ref=ring.at[o % RB],
                    dst_ref=out_ref.at[rows],
                    send_sem=x_send_sems.at[o],
                    recv_sem=x_recv_sems.at[o],
                    device_id=(other_x, my_y),
                    device_id_type=pl.DeviceIdType.MESH,
                )
                fwd = pltpu.make_async_remote_copy(
                    src_ref=out_ref.at[rows],
                    dst_ref=out_ref.at[rows],
                    send_sem=y_send_sems.at[o],
                    recv_sem=y_recv_sems.at[o],
                    device_id=(my_x, other_y),
                    device_id_type=pl.DeviceIdType.MESH,
                )
                y_sends[o] = fwd

                @pl.when(o < ns)
                def _(rcv=rcv, fwd=fwd):
                    rcv.wait_recv()
                    fwd.start()

            oy = c - LEAD - 1
            if 0 <= oy < SPLIT:
                rows_y = pl.ds(base_rem + chunk_id(oy, other_y) * r, r)
                yrcv = pltpu.make_async_remote_copy(
                    src_ref=ring.at[oy % RB],
                    dst_ref=out_ref.at[rows_y],
                    send_sem=y_send_sems.at[oy],
                    recv_sem=y_recv_sems.at[oy],
                    device_id=(my_x, other_y),
                    device_id_type=pl.DeviceIdType.MESH,
                )

                @pl.when(oy < ns_o)
                def _(yrcv=yrcv):
                    yrcv.wait_recv()

        for c in range(NC - RB, NC):
            outs[c].wait()
        for c in range(max(0, SPLIT - RB), SPLIT):
            if c + RB >= NC:
                @pl.when(c < ns)
                def _(c=c):
                    x_sends[c].wait_send()
        for o in range(SPLIT):
            @pl.when(o < ns)
            def _(o=o):
                y_sends[o].wait_send()

    return pl.pallas_call(
        body,
        out_shape=jax.ShapeDtypeStruct((2 * m, n), jnp.bfloat16),
        in_specs=[pl.BlockSpec(memory_space=pl.ANY)],
        out_specs=pl.BlockSpec(memory_space=pl.ANY),
        scratch_shapes=[
            pltpu.VMEM((S, m // NC, n), jnp.float32),
            pltpu.VMEM((RB, m // NC, n), jnp.bfloat16),
            pltpu.SemaphoreType.DMA((S,)),
            pltpu.SemaphoreType.DMA((NC,)),
            pltpu.SemaphoreType.DMA((SPLIT,)),
            pltpu.SemaphoreType.DMA((SPLIT,)),
            pltpu.SemaphoreType.DMA((SPLIT,)),
            pltpu.SemaphoreType.DMA((SPLIT,)),
        ],
        compiler_params=pltpu.CompilerParams(
            collective_id=0, vmem_limit_bytes=100 * 1024 * 1024
        ),
    )(x)


# device time: 476085 ns/iter; 1.1294x vs baseline; 1.1294x over previous
import jax
import jax.numpy as jnp
from jax import lax
from jax.experimental import pallas as pl
from jax.experimental.pallas import tpu as pltpu

HK = 16
LK = 2 * HK
S = 3
S2 = 2
SB = 6
LEAD = 3


def kernel(x):
    m, n = x.shape
    h = m // 2
    r = h // HK

    def body(x_ref, out_ref, stage, arena, rot, xarena, in_sems, out_sems,
             out2_sems, x_send_sems, x_recv_sems, y_send_sems, y_recv_sems):
        my_x = lax.axis_index("x")
        my_y = lax.axis_index("y")
        other_x = 1 - my_x
        other_y = 1 - my_y
        base_mine = my_x * m
        base_rem = other_x * m

        def chunk_rows(c):
            half = my_y if c < HK else other_y
            return half * h + (c % HK) * r

        barrier_sem = pltpu.get_barrier_semaphore()
        for nbr in [(other_x, my_y), (my_x, other_y)]:
            pl.semaphore_signal(
                barrier_sem, inc=1,
                device_id=nbr, device_id_type=pl.DeviceIdType.MESH,
            )
        pl.semaphore_wait(barrier_sem, 2)

        def stage_in(c):
            cp = pltpu.make_async_copy(
                x_ref.at[pl.ds(chunk_rows(c), r)],
                stage.at[c % S],
                in_sems.at[c % S],
            )
            cp.start()
            return cp

        def recv_step(c):
            rows = pl.ds(base_rem + my_y * h + c * r, r)
            recv = pltpu.make_async_remote_copy(
                src_ref=xarena.at[c],
                dst_ref=xarena.at[c],
                send_sem=x_send_sems.at[c],
                recv_sem=x_recv_sems.at[c],
                device_id=(other_x, my_y),
                device_id_type=pl.DeviceIdType.MESH,
            )
            recv.wait_recv()
            f = pltpu.make_async_remote_copy(
                src_ref=xarena.at[c],
                dst_ref=out_ref.at[rows],
                send_sem=y_send_sems.at[c],
                recv_sem=y_recv_sems.at[c],
                device_id=(my_x, other_y),
                device_id_type=pl.DeviceIdType.MESH,
            )
            f.start()
            lc = pltpu.make_async_copy(
                xarena.at[c], out_ref.at[rows], out2_sems.at[c]
            )
            lc.start()
            return f, lc

        ins = [None] * LK
        outs = [None] * LK
        x_sends = []
        y_sends = []
        x_places = []
        for c in range(S):
            ins[c] = stage_in(c)

        for c in range(LK):
            ins[c].wait()
            if c < HK:
                if c >= SB:
                    x_sends[c - SB].wait_send()
                    outs[c - SB].wait()
                arena[c % SB] = stage[c % S].astype(jnp.bfloat16)
                src = arena.at[c % SB]
            else:
                j = c - HK
                if j >= S2:
                    outs[HK + j - S2].wait()
                rot[j % S2] = stage[c % S].astype(jnp.bfloat16)
                src = rot.at[j % S2]
            nxt = c + S
            if nxt < LK:
                ins[nxt] = stage_in(nxt)
            outs[c] = pltpu.make_async_copy(
                src, out_ref.at[pl.ds(base_mine + chunk_rows(c), r)],
                out_sems.at[c],
            )
            outs[c].start()
            if c < HK:
                s = pltpu.make_async_remote_copy(
                    src_ref=arena.at[c % SB],
                    dst_ref=xarena.at[c],
                    send_sem=x_send_sems.at[c],
                    recv_sem=x_recv_sems.at[c],
                    device_id=(other_x, my_y),
                    device_id_type=pl.DeviceIdType.MESH,
                )
                s.start()
                x_sends.append(s)
            if LEAD <= c < HK + LEAD:
                f, lc = recv_step(c - LEAD)
                y_sends.append(f)
                x_places.append(lc)

        for c in range(HK):
            recv = pltpu.make_async_remote_copy(
                src_ref=xarena.at[c],
                dst_ref=out_ref.at[pl.ds(base_rem + other_y * h + c * r, r)],
                send_sem=y_send_sems.at[c],
                recv_sem=y_recv_sems.at[c],
                device_id=(my_x, other_y),
                device_id_type=pl.DeviceIdType.MESH,
            )
            recv.wait_recv()

        for c in range(LK - S2, LK):
            outs[c].wait()
        for c in range(HK - SB, HK):
            outs[c].wait()
        for lc in x_places:
            lc.wait()
        for s in x_sends[HK - SB:]:
            s.wait_send()
        for s in y_sends:
            s.wait_send()

    return pl.pallas_call(
        body,
        out_shape=jax.ShapeDtypeStruct((2 * m, n), jnp.bfloat16),
        in_specs=[pl.BlockSpec(memory_space=pl.ANY)],
        out_specs=pl.BlockSpec(memory_space=pl.ANY),
        scratch_shapes=[
            pltpu.VMEM((S, h // HK, n), jnp.float32),
            pltpu.VMEM((SB, h // HK, n), jnp.bfloat16),
            pltpu.VMEM((S2, h // HK, n), jnp.bfloat16),
            pltpu.VMEM((HK, h // HK, n), jnp.bfloat16),
            pltpu.SemaphoreType.DMA((S,)),
            pltpu.SemaphoreType.DMA((LK,)),
            pltpu.SemaphoreType.DMA((HK,)),
            pltpu.SemaphoreType.DMA((HK,)),
            pltpu.SemaphoreType.DMA((HK,)),
            pltpu.SemaphoreType.DMA((HK,)),
            pltpu.SemaphoreType.DMA((HK,)),
        ],
        compiler_params=pltpu.CompilerParams(
            collective_id=0, vmem_limit_bytes=100 * 1024 * 1024
        ),
    )(x)
